# baseline (speedup 1.0000x reference)
import functools

import jax
import jax.numpy as jnp
from jax.experimental import pallas as pl
from jax.experimental.pallas import tpu as pltpu

T = 2048
H = 768
EH = 768
EI = 32
ES = 4
K = 16
E_ALL = EI + ES

_C1 = 0.7978845608028654
_C2 = _C1 * 0.044715


def _router_kernel(x_ref, gwi_ref, gbi_ref, gws_ref, gbs_ref,
                   gi_ref, gs_ref, aux_ref):
    x = x_ref[...]
    li = jnp.dot(x, gwi_ref[...], preferred_element_type=jnp.float32)
    li = li + gbi_ref[...]
    lt = li.T
    m0 = jnp.max(lt, axis=0, keepdims=True)
    ex = jnp.exp(lt - m0)
    p = ex / jnp.sum(ex, axis=0, keepdims=True)

    iota = jax.lax.broadcasted_iota(jnp.int32, (EI, T), 0)
    rem = p
    sel = jnp.zeros((EI, T), dtype=jnp.bool_)
    for _ in range(K):
        m = jnp.max(rem, axis=0, keepdims=True)
        ismax = rem == m
        first = jnp.min(jnp.where(ismax, iota, EI), axis=0, keepdims=True)
        pick = iota == first
        sel = jnp.logical_or(sel, pick)
        rem = jnp.where(pick, -jnp.inf, rem)

    pv = jnp.where(sel, p, 0.0)
    gates_i = pv / jnp.sum(pv, axis=0, keepdims=True)
    gi_ref[...] = gates_i.T

    density = jnp.mean(sel.astype(jnp.float32), axis=1)
    mean_prob = jnp.mean(p, axis=1)
    aux = jnp.float32(EI) * jnp.sum(density * mean_prob)
    aux_ref[...] = jnp.reshape(aux, (1, 1))

    ls = jnp.dot(x, gws_ref[...], preferred_element_type=jnp.float32)
    ls = (ls + gbs_ref[...]).T
    ms = jnp.max(ls, axis=0, keepdims=True)
    es_ = jnp.exp(ls - ms)
    gs_ref[...] = (es_ / jnp.sum(es_, axis=0, keepdims=True)).T


def _mlp_contrib(num_e, e, x_ref, w1_ref, b1_ref, w2_ref, g_ref):
    onehot = (jax.lax.broadcasted_iota(jnp.int32, (num_e, 1), 0) == e
              ).astype(jnp.float32)
    g = jnp.dot(g_ref[...], onehot,
                preferred_element_type=jnp.float32)
    z = jnp.dot(x_ref[...], w1_ref[0], preferred_element_type=jnp.float32)
    z = z + b1_ref[0]
    t = jnp.tanh((_C2 * (z * z) + _C1) * z)
    hw = 0.5 * (z * g)
    h = hw + hw * t
    return jnp.dot(h, w2_ref[0], preferred_element_type=jnp.float32)


def _expert_i_kernel(x_ref, w1_ref, b1_ref, w2_ref, g_ref, out_ref):
    i = pl.program_id(0)
    oh = (jax.lax.broadcasted_iota(jnp.int32, (EI, 2), 0) ==
          2 * i + jax.lax.broadcasted_iota(jnp.int32, (EI, 2), 1)
          ).astype(jnp.float32)
    g2 = jnp.dot(g_ref[...], oh, preferred_element_type=jnp.float32)

    def mlp(k):
        z = jnp.dot(x_ref[...], w1_ref[k],
                    preferred_element_type=jnp.float32)
        z = z + b1_ref[k]
        t = jnp.tanh((_C2 * (z * z) + _C1) * z)
        hw = 0.5 * (z * g2[:, k:k + 1])
        h = hw + hw * t
        return jnp.dot(h, w2_ref[k], preferred_element_type=jnp.float32)

    y = mlp(0) + mlp(1)

    @pl.when(i == 0)
    def _():
        out_ref[...] = y

    @pl.when(i > 0)
    def _():
        out_ref[...] += y


def _expert_s_kernel(prev_ref, x_ref, w1_ref, b1_ref, w2_ref, g_ref,
                     gc_ref, b2c_ref, out_ref):
    e = pl.program_id(0)
    y = _mlp_contrib(ES, e, x_ref, w1_ref, b1_ref, w2_ref, g_ref)

    @pl.when(e == 0)
    def _():
        out_ref[...] = prev_ref[...] + y

    @pl.when(e > 0)
    def _():
        out_ref[...] += y

    @pl.when(e == ES - 1)
    def _():
        out_ref[...] += jnp.dot(gc_ref[...], b2c_ref[...],
                                preferred_element_type=jnp.float32)


@jax.jit
def kernel(x, gw_i, gb_i, w1_i, b1_i, w2_i, b2_i,
           gw_s, gb_s, w1_s, b1_s, w2_s, b2_s):
    gates_i, gates_s, aux = pl.pallas_call(
        _router_kernel,
        out_shape=(
            jax.ShapeDtypeStruct((T, EI), jnp.float32),
            jax.ShapeDtypeStruct((T, ES), jnp.float32),
            jax.ShapeDtypeStruct((1, 1), jnp.float32),
        ),
    )(x, gw_i, gb_i.reshape(1, EI), gw_s, gb_s.reshape(1, ES))

    out_i = pl.pallas_call(
        _expert_i_kernel,
        grid=(EI // 2,),
        in_specs=[
            pl.BlockSpec((T, H), lambda e: (0, 0)),
            pl.BlockSpec((2, H, EH), lambda e: (e, 0, 0)),
            pl.BlockSpec((2, 1, EH), lambda e: (e, 0, 0)),
            pl.BlockSpec((2, EH, H), lambda e: (e, 0, 0)),
            pl.BlockSpec((T, EI), lambda e: (0, 0)),
        ],
        out_specs=pl.BlockSpec((T, H), lambda e: (0, 0)),
        out_shape=jax.ShapeDtypeStruct((T, H), jnp.float32),
    )(x, w1_i, b1_i.reshape(EI, 1, EH), w2_i, gates_i)

    gc = jnp.concatenate([gates_i, gates_s], axis=1)
    b2c = jnp.concatenate([b2_i, b2_s], axis=0)

    out = pl.pallas_call(
        _expert_s_kernel,
        grid=(ES,),
        in_specs=[
            pl.BlockSpec((T, H), lambda e: (0, 0)),
            pl.BlockSpec((T, H), lambda e: (0, 0)),
            pl.BlockSpec((1, H, EH), lambda e: (e, 0, 0)),
            pl.BlockSpec((1, 1, EH), lambda e: (e, 0, 0)),
            pl.BlockSpec((1, EH, H), lambda e: (e, 0, 0)),
            pl.BlockSpec((T, ES), lambda e: (0, 0)),
            pl.BlockSpec((T, E_ALL), lambda e: (0, 0)),
            pl.BlockSpec((E_ALL, H), lambda e: (0, 0)),
        ],
        out_specs=pl.BlockSpec((T, H), lambda e: (0, 0)),
        out_shape=jax.ShapeDtypeStruct((T, H), jnp.float32),
        input_output_aliases={0: 0},
    )(out_i, x, w1_s, b1_s.reshape(ES, 1, EH), w2_s, gates_s, gc, b2c)

    return out, aux[0, 0]


# speedup vs baseline: 1.0812x; 1.0812x over previous
import jax
import jax.numpy as jnp
from jax.experimental import pallas as pl
from jax.experimental.pallas import tpu as pltpu

T = 2048
H = 768
EH = 768
EI = 32
ES = 4
K = 16

_C1 = 0.7978845608028654
_C2 = _C1 * 0.044715


def _topk_router(x, gw_ref, gb_ref):
    li = jnp.dot(x, gw_ref[...], preferred_element_type=jnp.float32)
    li = li + gb_ref[...]
    lt = li.T
    m0 = jnp.max(lt, axis=0, keepdims=True)
    ex = jnp.exp(lt - m0)
    p = ex / jnp.sum(ex, axis=0, keepdims=True)

    iota = jax.lax.broadcasted_iota(jnp.int32, (EI, T), 0)
    rem = p
    sel = jnp.zeros((EI, T), dtype=jnp.bool_)
    for _ in range(K):
        m = jnp.max(rem, axis=0, keepdims=True)
        ismax = rem == m
        first = jnp.min(jnp.where(ismax, iota, EI), axis=0, keepdims=True)
        pick = iota == first
        sel = jnp.logical_or(sel, pick)
        rem = jnp.where(pick, -jnp.inf, rem)

    pv = jnp.where(sel, p, 0.0)
    gates = (pv / jnp.sum(pv, axis=0, keepdims=True)).T

    density = jnp.mean(sel.astype(jnp.float32), axis=1)
    mean_prob = jnp.mean(p, axis=1)
    aux = jnp.float32(EI) * jnp.sum(density * mean_prob)
    return gates, aux


def _mlp_contrib(num_e, e, x, w1_ref, b1_ref, w2_ref, g_ref):
    onehot = (jax.lax.broadcasted_iota(jnp.int32, (num_e, 1), 0) == e
              ).astype(jnp.float32)
    g = jnp.dot(g_ref[...], onehot,
                preferred_element_type=jnp.float32)
    z = jnp.dot(x, w1_ref[0], preferred_element_type=jnp.float32)
    z = z + b1_ref[0]
    t = jnp.tanh((_C2 * (z * z) + _C1) * z)
    hw = 0.5 * (z * g)
    h = hw + hw * t
    return jnp.dot(h, w2_ref[0], preferred_element_type=jnp.float32)


def _expert_i_kernel(x_ref, gw_ref, gb_ref, w1_ref, b1_ref, w2_ref,
                     out_ref, gi_ref, aux_ref, gsc_ref):
    e = pl.program_id(0)

    @pl.when(e == 0)
    def _():
        gates, aux = _topk_router(x_ref[...], gw_ref, gb_ref)
        gsc_ref[...] = gates
        gi_ref[...] = gates
        aux_ref[...] = jnp.reshape(aux, (1, 1))

    y = _mlp_contrib(EI, e, x_ref[...], w1_ref, b1_ref, w2_ref, gsc_ref)

    @pl.when(e == 0)
    def _():
        out_ref[...] = y

    @pl.when(e > 0)
    def _():
        out_ref[...] += y


def _expert_s_kernel(prev_ref, x_ref, gw_ref, gb_ref, w1_ref, b1_ref,
                     w2_ref, gi_ref, b2i_ref, b2s_ref, out_ref, gsc_ref):
    e = pl.program_id(0)

    @pl.when(e == 0)
    def _():
        ls = jnp.dot(x_ref[...], gw_ref[...],
                     preferred_element_type=jnp.float32)
        ls = (ls + gb_ref[...]).T
        ms = jnp.max(ls, axis=0, keepdims=True)
        ex = jnp.exp(ls - ms)
        gsc_ref[...] = (ex / jnp.sum(ex, axis=0, keepdims=True)).T

    y = _mlp_contrib(ES, e, x_ref[...], w1_ref, b1_ref, w2_ref, gsc_ref)

    @pl.when(e == 0)
    def _():
        out_ref[...] = prev_ref[...] + y

    @pl.when(e > 0)
    def _():
        out_ref[...] += y

    @pl.when(e == ES - 1)
    def _():
        out_ref[...] += (
            jnp.dot(gi_ref[...], b2i_ref[...],
                    preferred_element_type=jnp.float32) +
            jnp.dot(gsc_ref[...], b2s_ref[...],
                    preferred_element_type=jnp.float32))


@jax.jit
def kernel(x, gw_i, gb_i, w1_i, b1_i, w2_i, b2_i,
           gw_s, gb_s, w1_s, b1_s, w2_s, b2_s):
    out_i, gates_i, aux = pl.pallas_call(
        _expert_i_kernel,
        grid=(EI,),
        in_specs=[
            pl.BlockSpec((T, H), lambda e: (0, 0)),
            pl.BlockSpec((H, EI), lambda e: (0, 0)),
            pl.BlockSpec((1, EI), lambda e: (0, 0)),
            pl.BlockSpec((1, H, EH), lambda e: (e, 0, 0)),
            pl.BlockSpec((1, 1, EH), lambda e: (e, 0, 0)),
            pl.BlockSpec((1, EH, H), lambda e: (e, 0, 0)),
        ],
        out_specs=(
            pl.BlockSpec((T, H), lambda e: (0, 0)),
            pl.BlockSpec((T, EI), lambda e: (0, 0)),
            pl.BlockSpec((1, 1), lambda e: (0, 0)),
        ),
        out_shape=(
            jax.ShapeDtypeStruct((T, H), jnp.float32),
            jax.ShapeDtypeStruct((T, EI), jnp.float32),
            jax.ShapeDtypeStruct((1, 1), jnp.float32),
        ),
        scratch_shapes=[pltpu.VMEM((T, EI), jnp.float32)],
    )(x, gw_i, gb_i.reshape(1, EI), w1_i,
      b1_i.reshape(EI, 1, EH), w2_i)

    out = pl.pallas_call(
        _expert_s_kernel,
        grid=(ES,),
        in_specs=[
            pl.BlockSpec((T, H), lambda e: (0, 0)),
            pl.BlockSpec((T, H), lambda e: (0, 0)),
            pl.BlockSpec((H, ES), lambda e: (0, 0)),
            pl.BlockSpec((1, ES), lambda e: (0, 0)),
            pl.BlockSpec((1, H, EH), lambda e: (e, 0, 0)),
            pl.BlockSpec((1, 1, EH), lambda e: (e, 0, 0)),
            pl.BlockSpec((1, EH, H), lambda e: (e, 0, 0)),
            pl.BlockSpec((T, EI), lambda e: (0, 0)),
            pl.BlockSpec((EI, H), lambda e: (0, 0)),
            pl.BlockSpec((ES, H), lambda e: (0, 0)),
        ],
        out_specs=pl.BlockSpec((T, H), lambda e: (0, 0)),
        out_shape=jax.ShapeDtypeStruct((T, H), jnp.float32),
        scratch_shapes=[pltpu.VMEM((T, ES), jnp.float32)],
        input_output_aliases={0: 0},
    )(out_i, x, gw_s, gb_s.reshape(1, ES), w1_s,
      b1_s.reshape(ES, 1, EH), w2_s, gates_i, b2_i, b2_s)

    return out, aux[0, 0]


# speedup vs baseline: 1.1040x; 1.0211x over previous
import jax
import jax.numpy as jnp
from jax.experimental import pallas as pl
from jax.experimental.pallas import tpu as pltpu

T = 2048
H = 768
EH = 768
EI = 32
ES = 4
K = 16

_C1 = 0.7978845608028654
_C2 = _C1 * 0.044715


def _topk_router(x, gw_ref, gb_ref):
    li = jnp.dot(x, gw_ref[...], preferred_element_type=jnp.float32)
    li = li + gb_ref[...]
    lt = li.T
    m0 = jnp.max(lt, axis=0, keepdims=True)
    ex = jnp.exp(lt - m0)
    p = ex / jnp.sum(ex, axis=0, keepdims=True)

    iota = jax.lax.broadcasted_iota(jnp.int32, (EI, T), 0)
    rem = p
    sel = jnp.zeros((EI, T), dtype=jnp.bool_)
    for _ in range(K):
        m = jnp.max(rem, axis=0, keepdims=True)
        ismax = rem == m
        first = jnp.min(jnp.where(ismax, iota, EI), axis=0, keepdims=True)
        pick = iota == first
        sel = jnp.logical_or(sel, pick)
        rem = jnp.where(pick, -jnp.inf, rem)

    pv = jnp.where(sel, p, 0.0)
    gates = (pv / jnp.sum(pv, axis=0, keepdims=True)).T

    density = jnp.mean(sel.astype(jnp.float32), axis=1)
    mean_prob = jnp.mean(p, axis=1)
    aux = jnp.float32(EI) * jnp.sum(density * mean_prob)
    return gates, aux


def _mlp_contrib(num_e, e, x, w1_ref, b1_ref, w2_ref, g_ref):
    onehot = (jax.lax.broadcasted_iota(jnp.int32, (num_e, 1), 0) == e
              ).astype(jnp.float32)
    g = jnp.dot(g_ref[...], onehot,
                preferred_element_type=jnp.float32)
    gh = 0.5 * g
    z = jnp.dot(x, w1_ref[0], preferred_element_type=jnp.float32)
    z = z + b1_ref[0]
    t = jnp.tanh((_C2 * (z * z) + _C1) * z)
    hw = z * gh
    h = hw + hw * t
    return jnp.dot(h, w2_ref[0], preferred_element_type=jnp.float32)


def _expert_i_kernel(x_ref, gw_ref, gb_ref, w1_ref, b1_ref, w2_ref,
                     out_ref, gi_ref, aux_ref, gsc_ref):
    e = pl.program_id(0)

    @pl.when(e == 0)
    def _():
        gates, aux = _topk_router(x_ref[...], gw_ref, gb_ref)
        gsc_ref[...] = gates
        gi_ref[...] = gates
        aux_ref[...] = jnp.reshape(aux, (1, 1))

    y = _mlp_contrib(EI, e, x_ref[...], w1_ref, b1_ref, w2_ref, gsc_ref)

    @pl.when(e == 0)
    def _():
        out_ref[...] = y

    @pl.when(e > 0)
    def _():
        out_ref[...] += y


def _expert_s_kernel(prev_ref, x_ref, gw_ref, gb_ref, w1_ref, b1_ref,
                     w2_ref, gi_ref, b2i_ref, b2s_ref, out_ref, gsc_ref):
    e = pl.program_id(0)

    @pl.when(e == 0)
    def _():
        ls = jnp.dot(x_ref[...], gw_ref[...],
                     preferred_element_type=jnp.float32)
        ls = (ls + gb_ref[...]).T
        ms = jnp.max(ls, axis=0, keepdims=True)
        ex = jnp.exp(ls - ms)
        gsc_ref[...] = (ex / jnp.sum(ex, axis=0, keepdims=True)).T

    y = _mlp_contrib(ES, e, x_ref[...], w1_ref, b1_ref, w2_ref, gsc_ref)

    @pl.when(e == 0)
    def _():
        out_ref[...] = prev_ref[...] + y

    @pl.when(e > 0)
    def _():
        out_ref[...] += y

    @pl.when(e == ES - 1)
    def _():
        out_ref[...] += (
            jnp.dot(gi_ref[...], b2i_ref[...],
                    preferred_element_type=jnp.float32) +
            jnp.dot(gsc_ref[...], b2s_ref[...],
                    preferred_element_type=jnp.float32))


@jax.jit
def kernel(x, gw_i, gb_i, w1_i, b1_i, w2_i, b2_i,
           gw_s, gb_s, w1_s, b1_s, w2_s, b2_s):
    out_i, gates_i, aux = pl.pallas_call(
        _expert_i_kernel,
        grid=(EI,),
        in_specs=[
            pl.BlockSpec((T, H), lambda e: (0, 0)),
            pl.BlockSpec((H, EI), lambda e: (0, 0)),
            pl.BlockSpec((1, EI), lambda e: (0, 0)),
            pl.BlockSpec((1, H, EH), lambda e: (e, 0, 0)),
            pl.BlockSpec((1, 1, EH), lambda e: (e, 0, 0)),
            pl.BlockSpec((1, EH, H), lambda e: (e, 0, 0)),
        ],
        out_specs=(
            pl.BlockSpec((T, H), lambda e: (0, 0)),
            pl.BlockSpec((T, EI), lambda e: (0, 0)),
            pl.BlockSpec((1, 1), lambda e: (0, 0)),
        ),
        out_shape=(
            jax.ShapeDtypeStruct((T, H), jnp.float32),
            jax.ShapeDtypeStruct((T, EI), jnp.float32),
            jax.ShapeDtypeStruct((1, 1), jnp.float32),
        ),
        scratch_shapes=[pltpu.VMEM((T, EI), jnp.float32)],
    )(x, gw_i, gb_i.reshape(1, EI), w1_i,
      b1_i.reshape(EI, 1, EH), w2_i)

    out = pl.pallas_call(
        _expert_s_kernel,
        grid=(ES,),
        in_specs=[
            pl.BlockSpec((T, H), lambda e: (0, 0)),
            pl.BlockSpec((T, H), lambda e: (0, 0)),
            pl.BlockSpec((H, ES), lambda e: (0, 0)),
            pl.BlockSpec((1, ES), lambda e: (0, 0)),
            pl.BlockSpec((1, H, EH), lambda e: (e, 0, 0)),
            pl.BlockSpec((1, 1, EH), lambda e: (e, 0, 0)),
            pl.BlockSpec((1, EH, H), lambda e: (e, 0, 0)),
            pl.BlockSpec((T, EI), lambda e: (0, 0)),
            pl.BlockSpec((EI, H), lambda e: (0, 0)),
            pl.BlockSpec((ES, H), lambda e: (0, 0)),
        ],
        out_specs=pl.BlockSpec((T, H), lambda e: (0, 0)),
        out_shape=jax.ShapeDtypeStruct((T, H), jnp.float32),
        scratch_shapes=[pltpu.VMEM((T, ES), jnp.float32)],
        input_output_aliases={0: 0},
    )(out_i, x, gw_s, gb_s.reshape(1, ES), w1_s,
      b1_s.reshape(ES, 1, EH), w2_s, gates_i, b2_i, b2_s)

    return out, aux[0, 0]
